# baseline (speedup 1.0000x reference)
import functools

import jax
import jax.numpy as jnp
from jax import lax
from jax.experimental import pallas as pl
from jax.experimental.pallas import tpu as pltpu
from jax.experimental.pallas import tpu_sc as plsc

B = 4096
DIN = 1024
H = 4
K_DIM = 256
HALF = K_DIM // 2
NK = 512
KNN = 32
DOUT = 512

TB = 512
NW = 32
TPW = B // NW
OUT_CHUNK = 8
LANES = 16
NQ = 2
QROWS = (H * KNN) // NQ
NCH = DOUT // LANES

CAND_W = [KNN // (i + 1) for i in range(KNN)]
CAND_TOT = sum(CAND_W)
CAND_PAD = 128 - CAND_TOT


def _topk(s, sidx=None):
    rows, n = s.shape
    iota_n = lax.broadcasted_iota(jnp.int32, (rows, n), 1)
    iota_k = lax.broadcasted_iota(jnp.int32, (rows, KNN), 1)
    neg_inf = jnp.float32(-jnp.inf)

    def body(t, carry):
        cur, vo, io = carry
        m = jnp.max(cur, axis=1, keepdims=True)
        pos = jnp.min(jnp.where(cur == m, iota_n, n), axis=1, keepdims=True)
        hit = iota_n == pos
        if sidx is None:
            isel = pos
        else:
            isel = jnp.max(jnp.where(hit, sidx, -1), axis=1, keepdims=True)
        cur = jnp.where(hit, neg_inf, cur)
        vo = jnp.where(iota_k == t, m, vo)
        io = jnp.where(iota_k == t, isel, io)
        return cur, vo, io

    _, vo, io = lax.fori_loop(
        0, KNN, body,
        (s, jnp.zeros((rows, KNN), jnp.float32),
         jnp.zeros((rows, KNN), jnp.int32)))
    return vo, io


def _tc_kernel(x_ref, wq_ref, bq_ref, keys_ref, idx_ref, w_ref):
    xb = x_ref[...]
    idx_parts = []
    w_parts = []
    for h in range(H):
        q = jnp.dot(xb, wq_ref[h], preferred_element_type=jnp.float32)
        q = q + bq_ref[h][None, :]
        q1 = q[:, :HALF]
        q2 = q[:, HALF:]
        s1 = lax.dot_general(q1, keys_ref[h, 0], (((1,), (1,)), ((), ())),
                             preferred_element_type=jnp.float32)
        s2 = lax.dot_general(q2, keys_ref[h, 1], (((1,), (1,)), ((), ())),
                             preferred_element_type=jnp.float32)
        sc1, i1 = _topk(s1)
        sc2, i2 = _topk(s2)
        comb = jnp.concatenate(
            [sc1[:, i:i + 1] + sc2[:, :w] for i, w in enumerate(CAND_W)]
            + [jnp.full((TB, CAND_PAD), -jnp.inf, jnp.float32)], axis=1)
        combidx = jnp.concatenate(
            [i1[:, i:i + 1] * NK + i2[:, :w] for i, w in enumerate(CAND_W)]
            + [jnp.zeros((TB, CAND_PAD), jnp.int32)], axis=1)
        scv, scidx = _topk(comb, combidx)
        e = jnp.exp(scv - scv[:, 0:1])
        w = e / jnp.sum(e, axis=1, keepdims=True)
        idx_parts.append(scidx)
        w_parts.append(w)
    idx_ref[...] = jnp.concatenate(idx_parts, axis=1)
    w_ref[...] = jnp.concatenate(w_parts, axis=1)


def _tc_stage(x, Wq, bq, keys):
    return pl.pallas_call(
        _tc_kernel,
        grid=(B // TB,),
        in_specs=[
            pl.BlockSpec((TB, DIN), lambda i: (i, 0)),
            pl.BlockSpec((H, DIN, K_DIM), lambda i: (0, 0, 0)),
            pl.BlockSpec((H, K_DIM), lambda i: (0, 0)),
            pl.BlockSpec((H, 2, NK, HALF), lambda i: (0, 0, 0, 0)),
        ],
        out_specs=[
            pl.BlockSpec((TB, H * KNN), lambda i: (i, 0)),
            pl.BlockSpec((TB, H * KNN), lambda i: (i, 0)),
        ],
        out_shape=[
            jax.ShapeDtypeStruct((B, H * KNN), jnp.int32),
            jax.ShapeDtypeStruct((B, H * KNN), jnp.float32),
        ],
    )(x, Wq, bq, keys)


def _sc_body(values_hbm, idx_hbm, w_hbm, out_hbm,
             idx_v, w_v, rows0_v, rows1_v, out_v, acc_v, sem0, sem1):
    wid = lax.axis_index("s") * 2 + lax.axis_index("c")
    base = wid * TPW
    pltpu.sync_copy(idx_hbm.at[pl.ds(base * NQ, TPW * NQ)], idx_v)
    pltpu.sync_copy(w_hbm.at[pl.ds(base, TPW)], w_v)

    dnums = lax.GatherDimensionNumbers(
        offset_dims=(), collapsed_slice_dims=(0,), start_index_map=(0,))
    bufs = (rows0_v, rows1_v)
    sems = (sem0, sem1)

    def start(qs, h):
        pltpu.async_copy(values_hbm.at[idx_v.at[qs]], bufs[h], sems[h])

    def wait(h):
        pltpu.make_async_copy(values_hbm.at[idx_v.at[0]], bufs[h],
                              sems[h]).wait()

    def accum(t, h):
        def chunk(jc, carry):
            wchunk = w_v[t, pl.ds(h * QROWS + jc * LANES, LANES)]
            for l in range(LANES):
                wj = lax.gather(
                    wchunk, jnp.full((LANES, 1), l, jnp.int32), dnums,
                    slice_sizes=(1,),
                    mode=lax.GatherScatterMode.PROMISE_IN_BOUNDS)
                for c in range(NCH):
                    sl = pl.ds(c * LANES, LANES)
                    plsc.addupdate(acc_v.at[sl],
                                   wj * bufs[h][jc * LANES + l, sl])
            return carry

        lax.fori_loop(0, QROWS // LANES, chunk, 0)

    for h in range(NQ):
        start(h, h)
    zeros = jnp.zeros((LANES,), jnp.float32)

    def tok_body(tl, g):
        t = g * OUT_CHUNK + tl
        for c in range(NCH):
            acc_v[pl.ds(c * LANES, LANES)] = zeros
        for h in range(NQ):
            wait(h)
            accum(t, h)

            @pl.when(t < TPW - 1)
            def _():
                start(NQ * (t + 1) + h, h)
        for c in range(NCH):
            sl = pl.ds(c * LANES, LANES)
            out_v[tl, sl] = acc_v[sl]
        return g

    def group_body(g, _):
        lax.fori_loop(0, OUT_CHUNK, tok_body, g)
        pltpu.sync_copy(out_v,
                        out_hbm.at[pl.ds(base + g * OUT_CHUNK, OUT_CHUNK)])
        return 0

    lax.fori_loop(0, TPW // OUT_CHUNK, group_body, 0)


@functools.lru_cache(maxsize=1)
def _make_sc_stage():
    return pl.kernel(
        _sc_body,
        out_type=jax.ShapeDtypeStruct((B, DOUT), jnp.float32),
        mesh=plsc.VectorSubcoreMesh(core_axis_name="c", subcore_axis_name="s"),
        scratch_types=[
            pltpu.VMEM((NQ * TPW, QROWS), jnp.int32),
            pltpu.VMEM((TPW, H * KNN), jnp.float32),
            pltpu.VMEM((QROWS, DOUT), jnp.float32),
            pltpu.VMEM((QROWS, DOUT), jnp.float32),
            pltpu.VMEM((OUT_CHUNK, DOUT), jnp.float32),
            pltpu.VMEM((DOUT,), jnp.float32),
            pltpu.SemaphoreType.DMA,
            pltpu.SemaphoreType.DMA,
        ],
    )


def kernel(x, Wq, bq, keys, values):
    idx, w = _tc_stage(x, Wq, bq, keys)
    idxq = idx.reshape(NQ * B, QROWS)
    return _make_sc_stage()(values, idxq, w)


# speedup vs baseline: 2.1864x; 2.1864x over previous
import functools

import jax
import jax.numpy as jnp
from jax import lax
from jax.experimental import pallas as pl
from jax.experimental.pallas import tpu as pltpu
from jax.experimental.pallas import tpu_sc as plsc

B = 4096
DIN = 1024
H = 4
K_DIM = 256
HALF = K_DIM // 2
NK = 512
KNN = 32
DOUT = 512

TB = 512
NW = 32
TPW = B // NW
OUT_CHUNK = 8
LANES = 16
NQ = 2
QROWS = (H * KNN) // NQ
NCH = DOUT // LANES

CAND_W = [KNN // (i + 1) for i in range(KNN)]
CAND_TOT = sum(CAND_W)
CAND_PAD = 128 - CAND_TOT


def _topk(s, sidx=None):
    rows, n = s.shape
    iota_n = lax.broadcasted_iota(jnp.int32, (rows, n), 1)
    iota_k = lax.broadcasted_iota(jnp.int32, (rows, KNN), 1)
    neg_inf = jnp.float32(-jnp.inf)

    def body(t, carry):
        cur, vo, io = carry
        m = jnp.max(cur, axis=1, keepdims=True)
        pos = jnp.min(jnp.where(cur == m, iota_n, n), axis=1, keepdims=True)
        hit = iota_n == pos
        if sidx is None:
            isel = pos
        else:
            isel = jnp.max(jnp.where(hit, sidx, -1), axis=1, keepdims=True)
        cur = jnp.where(hit, neg_inf, cur)
        vo = jnp.where(iota_k == t, m, vo)
        io = jnp.where(iota_k == t, isel, io)
        return cur, vo, io

    _, vo, io = lax.fori_loop(
        0, KNN, body,
        (s, jnp.zeros((rows, KNN), jnp.float32),
         jnp.zeros((rows, KNN), jnp.int32)))
    return vo, io


def _tc_kernel(x_ref, wq_ref, bq_ref, keys_ref, idx_ref, w_ref):
    xb = x_ref[...]
    idx_parts = []
    w_parts = []
    for h in range(H):
        q = jnp.dot(xb, wq_ref[h], preferred_element_type=jnp.float32)
        q = q + bq_ref[h][None, :]
        q1 = q[:, :HALF]
        q2 = q[:, HALF:]
        s1 = lax.dot_general(q1, keys_ref[h, 0], (((1,), (1,)), ((), ())),
                             preferred_element_type=jnp.float32)
        s2 = lax.dot_general(q2, keys_ref[h, 1], (((1,), (1,)), ((), ())),
                             preferred_element_type=jnp.float32)
        sc1, i1 = _topk(s1)
        sc2, i2 = _topk(s2)
        comb = jnp.concatenate(
            [sc1[:, i:i + 1] + sc2[:, :w] for i, w in enumerate(CAND_W)]
            + [jnp.full((TB, CAND_PAD), -jnp.inf, jnp.float32)], axis=1)
        combidx = jnp.concatenate(
            [i1[:, i:i + 1] * NK + i2[:, :w] for i, w in enumerate(CAND_W)]
            + [jnp.zeros((TB, CAND_PAD), jnp.int32)], axis=1)
        scv, scidx = _topk(comb, combidx)
        e = jnp.exp(scv - scv[:, 0:1])
        w = e / jnp.sum(e, axis=1, keepdims=True)
        idx_parts.append(scidx)
        w_parts.append(w)
    idx_ref[...] = jnp.concatenate(idx_parts, axis=1)
    w_ref[...] = jnp.concatenate(w_parts, axis=1)


def _tc_stage(x, Wq, bq, keys):
    return pl.pallas_call(
        _tc_kernel,
        grid=(B // TB,),
        in_specs=[
            pl.BlockSpec((TB, DIN), lambda i: (i, 0)),
            pl.BlockSpec((H, DIN, K_DIM), lambda i: (0, 0, 0)),
            pl.BlockSpec((H, K_DIM), lambda i: (0, 0)),
            pl.BlockSpec((H, 2, NK, HALF), lambda i: (0, 0, 0, 0)),
        ],
        out_specs=[
            pl.BlockSpec((TB, H * KNN), lambda i: (i, 0)),
            pl.BlockSpec((TB, H * KNN), lambda i: (i, 0)),
        ],
        out_shape=[
            jax.ShapeDtypeStruct((B, H * KNN), jnp.int32),
            jax.ShapeDtypeStruct((B, H * KNN), jnp.float32),
        ],
    )(x, Wq, bq, keys)


def _sc_body(values_hbm, idx_hbm, w_hbm, out_hbm,
             idx_v, w_v, rows0_v, rows1_v, out_v, acc_v, sem0, sem1):
    wid = lax.axis_index("s") * 2 + lax.axis_index("c")
    base = wid * TPW
    pltpu.sync_copy(idx_hbm.at[pl.ds(base * NQ, TPW * NQ)], idx_v)
    pltpu.sync_copy(w_hbm.at[pl.ds(base, TPW)], w_v)

    dnums = lax.GatherDimensionNumbers(
        offset_dims=(), collapsed_slice_dims=(0,), start_index_map=(0,))
    bufs = (rows0_v, rows1_v)
    sems = (sem0, sem1)

    def start(qs, h):
        pltpu.async_copy(values_hbm.at[idx_v.at[qs]], bufs[h], sems[h])

    def wait(h):
        pltpu.make_async_copy(values_hbm.at[idx_v.at[0]], bufs[h],
                              sems[h]).wait()

    def accum(t, h):
        def chunk(jc, carry):
            wchunk = w_v[t, pl.ds(h * QROWS + jc * LANES, LANES)]
            ws = []
            for l in range(LANES):
                ws.append(lax.gather(
                    wchunk, jnp.full((LANES, 1), l, jnp.int32), dnums,
                    slice_sizes=(1,),
                    mode=lax.GatherScatterMode.PROMISE_IN_BOUNDS))
            for c in range(NCH):
                sl = pl.ds(c * LANES, LANES)
                p = [ws[l] * bufs[h][jc * LANES + l, sl]
                     for l in range(LANES)]
                while len(p) > 1:
                    p = [p[i] + p[i + 1] for i in range(0, len(p), 2)]
                acc_v[sl] = acc_v[sl] + p[0]
            return carry

        lax.fori_loop(0, QROWS // LANES, chunk, 0)

    for h in range(NQ):
        start(h, h)
    zeros = jnp.zeros((LANES,), jnp.float32)

    def tok_body(tl, g):
        t = g * OUT_CHUNK + tl
        for c in range(NCH):
            acc_v[pl.ds(c * LANES, LANES)] = zeros
        for h in range(NQ):
            wait(h)
            accum(t, h)

            @pl.when(t < TPW - 1)
            def _():
                start(NQ * (t + 1) + h, h)
        for c in range(NCH):
            sl = pl.ds(c * LANES, LANES)
            out_v[tl, sl] = acc_v[sl]
        return g

    def group_body(g, _):
        lax.fori_loop(0, OUT_CHUNK, tok_body, g)
        pltpu.sync_copy(out_v,
                        out_hbm.at[pl.ds(base + g * OUT_CHUNK, OUT_CHUNK)])
        return 0

    lax.fori_loop(0, TPW // OUT_CHUNK, group_body, 0)


@functools.lru_cache(maxsize=1)
def _make_sc_stage():
    return pl.kernel(
        _sc_body,
        out_type=jax.ShapeDtypeStruct((B, DOUT), jnp.float32),
        mesh=plsc.VectorSubcoreMesh(core_axis_name="c", subcore_axis_name="s"),
        scratch_types=[
            pltpu.VMEM((NQ * TPW, QROWS), jnp.int32),
            pltpu.VMEM((TPW, H * KNN), jnp.float32),
            pltpu.VMEM((QROWS, DOUT), jnp.float32),
            pltpu.VMEM((QROWS, DOUT), jnp.float32),
            pltpu.VMEM((OUT_CHUNK, DOUT), jnp.float32),
            pltpu.VMEM((DOUT,), jnp.float32),
            pltpu.SemaphoreType.DMA,
            pltpu.SemaphoreType.DMA,
        ],
    )


def kernel(x, Wq, bq, keys, values):
    idx, w = _tc_stage(x, Wq, bq, keys)
    idxq = idx.reshape(NQ * B, QROWS)
    return _make_sc_stage()(values, idxq, w)


# speedup vs baseline: 2.3618x; 1.0802x over previous
import functools

import jax
import jax.numpy as jnp
from jax import lax
from jax.experimental import pallas as pl
from jax.experimental.pallas import tpu as pltpu
from jax.experimental.pallas import tpu_sc as plsc

B = 4096
DIN = 1024
H = 4
K_DIM = 256
HALF = K_DIM // 2
NK = 512
KNN = 32
DOUT = 512

TB = 1024
NW = 32
TPW = B // NW
OUT_CHUNK = 8
LANES = 16
NQ = 2
QROWS = (H * KNN) // NQ
NCH = DOUT // LANES

CAND_W = [KNN // (i + 1) for i in range(KNN)]
CAND_TOT = sum(CAND_W)
CAND_PAD = 128 - CAND_TOT


def _topk(s, sidx=None):
    rows, n = s.shape
    iota_n = lax.broadcasted_iota(jnp.int32, (rows, n), 1)
    iota_k = lax.broadcasted_iota(jnp.int32, (rows, KNN), 1)
    neg_inf = jnp.float32(-jnp.inf)

    def body(t, carry):
        cur, vo, io = carry
        m = jnp.max(cur, axis=1, keepdims=True)
        pos = jnp.min(jnp.where(cur == m, iota_n, n), axis=1, keepdims=True)
        hit = iota_n == pos
        if sidx is None:
            isel = pos
        else:
            isel = jnp.max(jnp.where(hit, sidx, -1), axis=1, keepdims=True)
        cur = jnp.where(hit, neg_inf, cur)
        vo = jnp.where(iota_k == t, m, vo)
        io = jnp.where(iota_k == t, isel, io)
        return cur, vo, io

    _, vo, io = lax.fori_loop(
        0, KNN, body,
        (s, jnp.zeros((rows, KNN), jnp.float32),
         jnp.zeros((rows, KNN), jnp.int32)))
    return vo, io


def _tc_kernel(x_ref, wq_ref, bq_ref, keys_ref, idx_ref, w_ref):
    xb = x_ref[...]
    idx_parts = []
    w_parts = []
    for h in range(H):
        q = jnp.dot(xb, wq_ref[h], preferred_element_type=jnp.float32)
        q = q + bq_ref[h][None, :]
        q1 = q[:, :HALF]
        q2 = q[:, HALF:]
        s1 = lax.dot_general(q1, keys_ref[h, 0], (((1,), (1,)), ((), ())),
                             preferred_element_type=jnp.float32)
        s2 = lax.dot_general(q2, keys_ref[h, 1], (((1,), (1,)), ((), ())),
                             preferred_element_type=jnp.float32)
        sc1, i1 = _topk(s1)
        sc2, i2 = _topk(s2)
        comb = jnp.concatenate(
            [sc1[:, i:i + 1] + sc2[:, :w] for i, w in enumerate(CAND_W)]
            + [jnp.full((TB, CAND_PAD), -jnp.inf, jnp.float32)], axis=1)
        combidx = jnp.concatenate(
            [i1[:, i:i + 1] * NK + i2[:, :w] for i, w in enumerate(CAND_W)]
            + [jnp.zeros((TB, CAND_PAD), jnp.int32)], axis=1)
        scv, scidx = _topk(comb, combidx)
        e = jnp.exp(scv - scv[:, 0:1])
        w = e / jnp.sum(e, axis=1, keepdims=True)
        idx_parts.append(scidx)
        w_parts.append(w)
    idx_ref[...] = jnp.concatenate(idx_parts, axis=1)
    w_ref[...] = jnp.concatenate(w_parts, axis=1)


def _tc_stage(x, Wq, bq, keys):
    return pl.pallas_call(
        _tc_kernel,
        grid=(B // TB,),
        in_specs=[
            pl.BlockSpec((TB, DIN), lambda i: (i, 0)),
            pl.BlockSpec((H, DIN, K_DIM), lambda i: (0, 0, 0)),
            pl.BlockSpec((H, K_DIM), lambda i: (0, 0)),
            pl.BlockSpec((H, 2, NK, HALF), lambda i: (0, 0, 0, 0)),
        ],
        out_specs=[
            pl.BlockSpec((TB, H * KNN), lambda i: (i, 0)),
            pl.BlockSpec((TB, H * KNN), lambda i: (i, 0)),
        ],
        out_shape=[
            jax.ShapeDtypeStruct((B, H * KNN), jnp.int32),
            jax.ShapeDtypeStruct((B, H * KNN), jnp.float32),
        ],
    )(x, Wq, bq, keys)


def _sc_body(values_hbm, idx_hbm, w_hbm, out_hbm,
             idx_v, w_v, rows0_v, rows1_v, out_v, acc_v, sem0, sem1):
    wid = lax.axis_index("s") * 2 + lax.axis_index("c")
    base = wid * TPW
    pltpu.sync_copy(idx_hbm.at[pl.ds(base * NQ, TPW * NQ)], idx_v)
    pltpu.sync_copy(w_hbm.at[pl.ds(base, TPW)], w_v)

    dnums = lax.GatherDimensionNumbers(
        offset_dims=(), collapsed_slice_dims=(0,), start_index_map=(0,))
    bufs = (rows0_v, rows1_v)
    sems = (sem0, sem1)

    def start(qs, h):
        pltpu.async_copy(values_hbm.at[idx_v.at[qs]], bufs[h], sems[h])

    def wait(h):
        pltpu.make_async_copy(values_hbm.at[idx_v.at[0]], bufs[h],
                              sems[h]).wait()

    def accum(t, h):
        def chunk(jc, carry):
            wchunk = w_v[t, pl.ds(h * QROWS + jc * LANES, LANES)]
            ws = []
            for l in range(LANES):
                ws.append(lax.gather(
                    wchunk, jnp.full((LANES, 1), l, jnp.int32), dnums,
                    slice_sizes=(1,),
                    mode=lax.GatherScatterMode.PROMISE_IN_BOUNDS))
            for c in range(NCH):
                sl = pl.ds(c * LANES, LANES)
                p = [ws[l] * bufs[h][jc * LANES + l, sl]
                     for l in range(LANES)]
                while len(p) > 1:
                    p = [p[i] + p[i + 1] for i in range(0, len(p), 2)]
                acc_v[sl] = acc_v[sl] + p[0]
            return carry

        lax.fori_loop(0, QROWS // LANES, chunk, 0)

    for h in range(NQ):
        start(h, h)
    zeros = jnp.zeros((LANES,), jnp.float32)

    def tok_body(tl, g):
        t = g * OUT_CHUNK + tl
        for c in range(NCH):
            acc_v[pl.ds(c * LANES, LANES)] = zeros
        for h in range(NQ):
            wait(h)
            accum(t, h)

            @pl.when(t < TPW - 1)
            def _():
                start(NQ * (t + 1) + h, h)
        for c in range(NCH):
            sl = pl.ds(c * LANES, LANES)
            out_v[tl, sl] = acc_v[sl]
        return g

    def group_body(g, _):
        lax.fori_loop(0, OUT_CHUNK, tok_body, g)
        pltpu.sync_copy(out_v,
                        out_hbm.at[pl.ds(base + g * OUT_CHUNK, OUT_CHUNK)])
        return 0

    lax.fori_loop(0, TPW // OUT_CHUNK, group_body, 0)


@functools.lru_cache(maxsize=1)
def _make_sc_stage():
    return pl.kernel(
        _sc_body,
        out_type=jax.ShapeDtypeStruct((B, DOUT), jnp.float32),
        mesh=plsc.VectorSubcoreMesh(core_axis_name="c", subcore_axis_name="s"),
        scratch_types=[
            pltpu.VMEM((NQ * TPW, QROWS), jnp.int32),
            pltpu.VMEM((TPW, H * KNN), jnp.float32),
            pltpu.VMEM((QROWS, DOUT), jnp.float32),
            pltpu.VMEM((QROWS, DOUT), jnp.float32),
            pltpu.VMEM((OUT_CHUNK, DOUT), jnp.float32),
            pltpu.VMEM((DOUT,), jnp.float32),
            pltpu.SemaphoreType.DMA,
            pltpu.SemaphoreType.DMA,
        ],
    )


def kernel(x, Wq, bq, keys, values):
    idx, w = _tc_stage(x, Wq, bq, keys)
    idxq = idx.reshape(NQ * B, QROWS)
    return _make_sc_stage()(values, idxq, w)
